# initial kernel scaffold (unmeasured)
import jax
import jax.numpy as jnp
from jax import lax
from jax.experimental import pallas as pl
from jax.experimental.pallas import tpu as pltpu

N_DEV = 4
N_TOK = 256
D_IN = 128
D_OUT = 256
N_EXP = 8
EXP_PER_DEV = N_EXP // N_DEV
ROWS_PER_DEV = N_TOK // N_DEV


def kernel(x, router_W, route_idx, expert_W, shared_W):
    def body(x_ref, router_ref, idx_ref, expW_ref, sharedW_ref,
             out_ref, partial_ref, comm_ref, send_sems, recv_sems):
        my_pos = lax.axis_index("i")

        barrier_sem = pltpu.get_barrier_semaphore()
        for d in range(1, N_DEV):
            peer = lax.rem(my_pos + d, N_DEV)
            pl.semaphore_signal(
                barrier_sem, inc=1,
                device_id=(peer,), device_id_type=pl.DeviceIdType.MESH,
            )
        pl.semaphore_wait(barrier_sem, N_DEV - 1)

        xf = x_ref[:, :]
        scores = jnp.dot(xf, router_ref[:, :],
                         preferred_element_type=jnp.float32)
        scores = scores - jnp.max(scores, axis=-1, keepdims=True)
        e_scores = jnp.exp(scores)
        probs = e_scores / jnp.sum(e_scores, axis=-1, keepdims=True)
        route = idx_ref[:, :]
        eight = lax.broadcasted_iota(jnp.int32, (N_TOK, N_EXP), 1)
        gate = jnp.sum(jnp.where(route == eight, probs, 0.0),
                       axis=-1, keepdims=True)

        x_bf = xf.astype(jnp.bfloat16)
        partial = jnp.zeros((N_TOK, D_OUT), jnp.float32)
        for le in range(EXP_PER_DEV):
            e = my_pos * EXP_PER_DEV + le
            y = jnp.dot(x_bf, expW_ref[le, :, :].astype(jnp.bfloat16),
                        preferred_element_type=jnp.float32)
            w = jnp.where(route[:, :1] == e, gate, 0.0)
            partial = partial + w * y
        partial_ref[:, :] = partial

        rdmas = []
        for d in range(1, N_DEV):
            q = lax.rem(my_pos + d, N_DEV)
            rdma = pltpu.make_async_remote_copy(
                src_ref=partial_ref.at[pl.ds(q * ROWS_PER_DEV, ROWS_PER_DEV), :],
                dst_ref=comm_ref.at[d - 1],
                send_sem=send_sems.at[d - 1],
                recv_sem=recv_sems.at[d - 1],
                device_id=(q,),
                device_id_type=pl.DeviceIdType.MESH,
            )
            rdma.start()
            rdmas.append(rdma)

        my_rows = pl.ds(my_pos * ROWS_PER_DEV, ROWS_PER_DEV)
        shared = jnp.dot(x_bf[my_rows, :].astype(jnp.bfloat16),
                         sharedW_ref[:, :].astype(jnp.bfloat16),
                         preferred_element_type=jnp.float32)
        acc = shared + partial_ref[my_rows, :]

        for d in range(1, N_DEV):
            rdmas[d - 1].wait_recv()
            acc = acc + comm_ref[d - 1, :, :]
        out_ref[:, :] = acc

        for d in range(1, N_DEV):
            rdmas[d - 1].wait_send()

    return pl.pallas_call(
        body,
        out_shape=jax.ShapeDtypeStruct((ROWS_PER_DEV, D_OUT), jnp.float32),
        in_specs=[
            pl.BlockSpec(memory_space=pltpu.VMEM),
            pl.BlockSpec(memory_space=pltpu.VMEM),
            pl.BlockSpec(memory_space=pltpu.VMEM),
            pl.BlockSpec(memory_space=pltpu.VMEM),
            pl.BlockSpec(memory_space=pltpu.VMEM),
        ],
        out_specs=pl.BlockSpec(memory_space=pltpu.VMEM),
        scratch_shapes=[
            pltpu.VMEM((N_TOK, D_OUT), jnp.float32),
            pltpu.VMEM((N_DEV - 1, ROWS_PER_DEV, D_OUT), jnp.float32),
            pltpu.SemaphoreType.DMA((N_DEV - 1,)),
            pltpu.SemaphoreType.DMA((N_DEV - 1,)),
        ],
        compiler_params=pltpu.CompilerParams(collective_id=0),
    )(x, router_W, route_idx, expert_W, shared_W)


# baseline (device time: 10319 ns/iter reference)
import jax
import jax.numpy as jnp
from jax import lax
from jax.experimental import pallas as pl
from jax.experimental.pallas import tpu as pltpu

N_DEV = 4
N_TOK = 256
D_IN = 128
D_OUT = 256
N_EXP = 8
EXP_PER_DEV = N_EXP // N_DEV
ROWS_PER_DEV = N_TOK // N_DEV


def kernel(x, router_W, route_idx, expert_W, shared_W):
    def body(x_ref, router_ref, idx_ref, expW_ref, sharedW_ref,
             out_ref, partial_ref, comm_ref, send_sems, recv_sems):
        my_pos = lax.axis_index("i")

        barrier_sem = pltpu.get_barrier_semaphore()
        for d in range(1, N_DEV):
            peer = lax.rem(my_pos + d, N_DEV)
            pl.semaphore_signal(
                barrier_sem, inc=1,
                device_id=(peer,), device_id_type=pl.DeviceIdType.MESH,
            )
        pl.semaphore_wait(barrier_sem, N_DEV - 1)

        xf = x_ref[:, :]
        scores = jnp.dot(xf, router_ref[:, :],
                         preferred_element_type=jnp.float32)
        scores = scores - jnp.max(scores, axis=-1, keepdims=True)
        e_scores = jnp.exp(scores)
        probs = e_scores / jnp.sum(e_scores, axis=-1, keepdims=True)
        route = idx_ref[:, :]
        eight = lax.broadcasted_iota(jnp.int32, (N_TOK, N_EXP), 1)
        gate = jnp.sum(jnp.where(route == eight, probs, 0.0),
                       axis=-1, keepdims=True)

        x_bf = xf.astype(jnp.bfloat16)
        partial = jnp.zeros((N_TOK, D_OUT), jnp.float32)
        for le in range(EXP_PER_DEV):
            e = my_pos * EXP_PER_DEV + le
            y = jnp.dot(x_bf, expW_ref[le, :, :].astype(jnp.bfloat16),
                        preferred_element_type=jnp.float32)
            w = jnp.where(route[:, :1] == e, gate, 0.0)
            partial = partial + w * y
        partial_ref[:, :] = partial

        rdmas = []
        for d in range(1, N_DEV):
            q = lax.rem(my_pos + d, N_DEV)
            rdma = pltpu.make_async_remote_copy(
                src_ref=partial_ref.at[pl.ds(q * ROWS_PER_DEV, ROWS_PER_DEV), :],
                dst_ref=comm_ref.at[d - 1],
                send_sem=send_sems.at[d - 1],
                recv_sem=recv_sems.at[d - 1],
                device_id=(q,),
                device_id_type=pl.DeviceIdType.MESH,
            )
            rdma.start()
            rdmas.append(rdma)

        my_rows = pl.ds(my_pos * ROWS_PER_DEV, ROWS_PER_DEV)
        shared = jnp.dot(x_ref[my_rows, :].astype(jnp.bfloat16),
                         sharedW_ref[:, :].astype(jnp.bfloat16),
                         preferred_element_type=jnp.float32)
        acc = shared + partial_ref[my_rows, :]

        for d in range(1, N_DEV):
            rdmas[d - 1].wait_recv()
            acc = acc + comm_ref[d - 1, :, :]
        out_ref[:, :] = acc

        for d in range(1, N_DEV):
            rdmas[d - 1].wait_send()

    return pl.pallas_call(
        body,
        out_shape=jax.ShapeDtypeStruct((ROWS_PER_DEV, D_OUT), jnp.float32),
        in_specs=[
            pl.BlockSpec(memory_space=pltpu.VMEM),
            pl.BlockSpec(memory_space=pltpu.VMEM),
            pl.BlockSpec(memory_space=pltpu.VMEM),
            pl.BlockSpec(memory_space=pltpu.VMEM),
            pl.BlockSpec(memory_space=pltpu.VMEM),
        ],
        out_specs=pl.BlockSpec(memory_space=pltpu.VMEM),
        scratch_shapes=[
            pltpu.VMEM((N_TOK, D_OUT), jnp.float32),
            pltpu.VMEM((N_DEV - 1, ROWS_PER_DEV, D_OUT), jnp.float32),
            pltpu.SemaphoreType.DMA((N_DEV - 1,)),
            pltpu.SemaphoreType.DMA((N_DEV - 1,)),
        ],
        compiler_params=pltpu.CompilerParams(collective_id=0),
    )(x, router_W, route_idx, expert_W, shared_W)


# device time: 9657 ns/iter; 1.0686x vs baseline; 1.0686x over previous
import jax
import jax.numpy as jnp
from jax import lax
from jax.experimental import pallas as pl
from jax.experimental.pallas import tpu as pltpu

N_DEV = 4
N_TOK = 256
D_IN = 128
D_OUT = 256
N_EXP = 8
EXP_PER_DEV = N_EXP // N_DEV
ROWS_PER_DEV = N_TOK // N_DEV


def kernel(x, router_W, route_idx, expert_W, shared_W):
    def body(x_ref, router_ref, idx_ref, expW_ref, sharedW_ref,
             out_ref, partial_ref, comm_ref, send_sems, recv_sems):
        my_pos = lax.axis_index("i")

        barrier_sem = pltpu.get_barrier_semaphore()
        for d in range(1, N_DEV):
            peer = lax.rem(my_pos + d, N_DEV)
            pl.semaphore_signal(
                barrier_sem, inc=1,
                device_id=(peer,), device_id_type=pl.DeviceIdType.MESH,
            )
        pl.semaphore_wait(barrier_sem, N_DEV - 1)

        xf = x_ref[:, :]
        scores = jnp.dot(xf, router_ref[:, :],
                         preferred_element_type=jnp.float32)
        scores = scores - jnp.max(scores, axis=-1, keepdims=True)
        e_scores = jnp.exp(scores)
        probs = e_scores / jnp.sum(e_scores, axis=-1, keepdims=True)
        route = idx_ref[:, :]
        eight = lax.broadcasted_iota(jnp.int32, (N_TOK, N_EXP), 1)
        gate = jnp.sum(jnp.where(route == eight, probs, 0.0),
                       axis=-1, keepdims=True)

        x_bf = xf.astype(jnp.bfloat16)
        partial = jnp.zeros((N_TOK, D_OUT), jnp.float32)
        for le in range(EXP_PER_DEV):
            e = my_pos * EXP_PER_DEV + le
            y = jnp.dot(x_bf, expW_ref[le, :, :].astype(jnp.bfloat16),
                        preferred_element_type=jnp.float32)
            w = jnp.where(route[:, :1] == e, gate, 0.0)
            partial = partial + w * y
        partial_ref[:, :] = partial.astype(jnp.bfloat16)

        rdmas = []
        for d in range(1, N_DEV):
            q = lax.rem(my_pos + d, N_DEV)
            rdma = pltpu.make_async_remote_copy(
                src_ref=partial_ref.at[pl.ds(q * ROWS_PER_DEV, ROWS_PER_DEV), :],
                dst_ref=comm_ref.at[d - 1],
                send_sem=send_sems.at[d - 1],
                recv_sem=recv_sems.at[d - 1],
                device_id=(q,),
                device_id_type=pl.DeviceIdType.MESH,
            )
            rdma.start()
            rdmas.append(rdma)

        my_rows = pl.ds(my_pos * ROWS_PER_DEV, ROWS_PER_DEV)
        shared = jnp.dot(x_ref[my_rows, :].astype(jnp.bfloat16),
                         sharedW_ref[:, :].astype(jnp.bfloat16),
                         preferred_element_type=jnp.float32)
        acc = shared + partial_ref[my_rows, :].astype(jnp.float32)

        for d in range(1, N_DEV):
            rdmas[d - 1].wait_recv()
            acc = acc + comm_ref[d - 1, :, :].astype(jnp.float32)
        out_ref[:, :] = acc

        for d in range(1, N_DEV):
            rdmas[d - 1].wait_send()

    return pl.pallas_call(
        body,
        out_shape=jax.ShapeDtypeStruct((ROWS_PER_DEV, D_OUT), jnp.float32),
        in_specs=[
            pl.BlockSpec(memory_space=pltpu.VMEM),
            pl.BlockSpec(memory_space=pltpu.VMEM),
            pl.BlockSpec(memory_space=pltpu.VMEM),
            pl.BlockSpec(memory_space=pltpu.VMEM),
            pl.BlockSpec(memory_space=pltpu.VMEM),
        ],
        out_specs=pl.BlockSpec(memory_space=pltpu.VMEM),
        scratch_shapes=[
            pltpu.VMEM((N_TOK, D_OUT), jnp.bfloat16),
            pltpu.VMEM((N_DEV - 1, ROWS_PER_DEV, D_OUT), jnp.bfloat16),
            pltpu.SemaphoreType.DMA((N_DEV - 1,)),
            pltpu.SemaphoreType.DMA((N_DEV - 1,)),
        ],
        compiler_params=pltpu.CompilerParams(collective_id=0),
    )(x, router_W, route_idx, expert_W, shared_W)


# device time: 9064 ns/iter; 1.1385x vs baseline; 1.0654x over previous
import jax
import jax.numpy as jnp
from jax import lax
from jax.experimental import pallas as pl
from jax.experimental.pallas import tpu as pltpu

N_DEV = 4
N_TOK = 256
D_IN = 128
D_OUT = 256
N_EXP = 8
EXP_PER_DEV = N_EXP // N_DEV
ROWS_PER_DEV = N_TOK // N_DEV


def kernel(x, router_W, route_idx, expert_W, shared_W):
    def body(x_ref, router_ref, idx_ref, expW_ref, sharedW_ref,
             out_ref, partial_ref, comm_ref, send_sems, recv_sems):
        my_pos = lax.axis_index("i")

        barrier_sem = pltpu.get_barrier_semaphore()
        for d in range(1, N_DEV):
            peer = lax.rem(my_pos + d, N_DEV)
            pl.semaphore_signal(
                barrier_sem, inc=1,
                device_id=(peer,), device_id_type=pl.DeviceIdType.MESH,
            )

        xf = x_ref[:, :]
        scores = jnp.dot(xf, router_ref[:, :],
                         preferred_element_type=jnp.float32)
        scores = scores - jnp.max(scores, axis=-1, keepdims=True)
        e_scores = jnp.exp(scores)
        probs = e_scores / jnp.sum(e_scores, axis=-1, keepdims=True)
        route = idx_ref[:, :]
        eight = lax.broadcasted_iota(jnp.int32, (N_TOK, N_EXP), 1)
        gate = jnp.sum(jnp.where(route == eight, probs, 0.0),
                       axis=-1, keepdims=True)

        x_bf = xf.astype(jnp.bfloat16)
        partial = jnp.zeros((N_TOK, D_OUT), jnp.float32)
        for le in range(EXP_PER_DEV):
            e = my_pos * EXP_PER_DEV + le
            y = jnp.dot(x_bf, expW_ref[le, :, :].astype(jnp.bfloat16),
                        preferred_element_type=jnp.float32)
            w = jnp.where(route[:, :1] == e, gate, 0.0)
            partial = partial + w * y
        partial_ref[:, :] = partial.astype(jnp.bfloat16)

        pl.semaphore_wait(barrier_sem, N_DEV - 1)

        rdmas = []
        for d in range(1, N_DEV):
            q = lax.rem(my_pos + d, N_DEV)
            rdma = pltpu.make_async_remote_copy(
                src_ref=partial_ref.at[pl.ds(q * ROWS_PER_DEV, ROWS_PER_DEV), :],
                dst_ref=comm_ref.at[d - 1],
                send_sem=send_sems.at[d - 1],
                recv_sem=recv_sems.at[d - 1],
                device_id=(q,),
                device_id_type=pl.DeviceIdType.MESH,
            )
            rdma.start()
            rdmas.append(rdma)

        my_rows = pl.ds(my_pos * ROWS_PER_DEV, ROWS_PER_DEV)
        shared = jnp.dot(x_ref[my_rows, :].astype(jnp.bfloat16),
                         sharedW_ref[:, :].astype(jnp.bfloat16),
                         preferred_element_type=jnp.float32)
        acc = shared + partial_ref[my_rows, :].astype(jnp.float32)

        for d in range(1, N_DEV):
            rdmas[d - 1].wait_recv()
            acc = acc + comm_ref[d - 1, :, :].astype(jnp.float32)
        out_ref[:, :] = acc

        for d in range(1, N_DEV):
            rdmas[d - 1].wait_send()

    return pl.pallas_call(
        body,
        out_shape=jax.ShapeDtypeStruct((ROWS_PER_DEV, D_OUT), jnp.float32),
        in_specs=[
            pl.BlockSpec(memory_space=pltpu.VMEM),
            pl.BlockSpec(memory_space=pltpu.VMEM),
            pl.BlockSpec(memory_space=pltpu.VMEM),
            pl.BlockSpec(memory_space=pltpu.VMEM),
            pl.BlockSpec(memory_space=pltpu.VMEM),
        ],
        out_specs=pl.BlockSpec(memory_space=pltpu.VMEM),
        scratch_shapes=[
            pltpu.VMEM((N_TOK, D_OUT), jnp.bfloat16),
            pltpu.VMEM((N_DEV - 1, ROWS_PER_DEV, D_OUT), jnp.bfloat16),
            pltpu.SemaphoreType.DMA((N_DEV - 1,)),
            pltpu.SemaphoreType.DMA((N_DEV - 1,)),
        ],
        compiler_params=pltpu.CompilerParams(collective_id=0),
    )(x, router_W, route_idx, expert_W, shared_W)
